# Initial kernel scaffold; baseline (speedup 1.0000x reference)
import functools

import jax
import jax.numpy as jnp
from jax import lax
from jax.experimental import pallas as pl
from jax.experimental.pallas import tpu as pltpu
from jax.experimental.pallas import tpu_sc as plsc

NSEG = 16384
NWORKERS = 32
CHUNK = 2048
LANES = 16


def _sc_partial_sums(cols, vals, per_tile, nchunk):
    mesh = plsc.VectorSubcoreMesh(core_axis_name="c", subcore_axis_name="s")

    @functools.partial(
        pl.kernel,
        out_type=jax.ShapeDtypeStruct((NWORKERS, NSEG), jnp.float32),
        mesh=mesh,
        scratch_types=[
            pltpu.VMEM((NSEG,), jnp.float32),
            pltpu.VMEM((CHUNK,), jnp.int32),
            pltpu.VMEM((CHUNK,), jnp.float32),
        ],
    )
    def sc_kernel(cols_hbm, vals_hbm, part_hbm, acc_v, idx_v, val_v):
        cid = lax.axis_index("c")
        sid = lax.axis_index("s")
        wid = sid * 2 + cid
        base = wid * per_tile

        zeros = jnp.zeros((LANES,), jnp.float32)

        def zero_body(i, _):
            acc_v[pl.ds(i * LANES, LANES)] = zeros
            return ()

        lax.fori_loop(0, NSEG // LANES, zero_body, ())

        def chunk_body(g, _):
            off = base + g * CHUNK
            pltpu.sync_copy(cols_hbm.at[pl.ds(off, CHUNK)], idx_v)
            pltpu.sync_copy(vals_hbm.at[pl.ds(off, CHUNK)], val_v)

            def inner(i, _):
                idx = idx_v[pl.ds(i * LANES, LANES)]
                v = val_v[pl.ds(i * LANES, LANES)]
                plsc.addupdate_scatter(acc_v, [idx], v)
                return ()

            lax.fori_loop(0, CHUNK // LANES, inner, ())
            return ()

        lax.fori_loop(0, nchunk, chunk_body, ())

        pltpu.sync_copy(acc_v, part_hbm.at[wid])

    return sc_kernel(cols, vals)


def _tc_reduce(partials):
    def body(p_ref, o_ref):
        o_ref[...] = jnp.sum(p_ref[...], axis=0, keepdims=True)

    return pl.pallas_call(
        body,
        out_shape=jax.ShapeDtypeStruct((1, NSEG), jnp.float32),
    )(partials)


def kernel(indices, values):
    cols = indices[1].astype(jnp.int32)
    vals = values.astype(jnp.float32)
    nnz = vals.shape[0]

    per_tile = -(-nnz // NWORKERS)
    per_tile = -(-per_tile // CHUNK) * CHUNK
    nchunk = per_tile // CHUNK
    total = per_tile * NWORKERS

    pad = total - nnz
    cols = jnp.pad(cols, (0, pad))
    vals = jnp.pad(vals, (0, pad))

    partials = _sc_partial_sums(cols, vals, per_tile, nchunk)
    return _tc_reduce(partials).reshape(NSEG)


# speedup vs baseline: 63.8884x; 63.8884x over previous
import functools

import jax
import jax.numpy as jnp
from jax import lax
from jax.experimental import pallas as pl
from jax.experimental.pallas import tpu as pltpu
from jax.experimental.pallas import tpu_sc as plsc

NSEG = 16384
NWORKERS = 32
CHUNK = 2048
LANES = 16


def _sc_partial_sums(cols, vals, per_tile, nchunk):
    mesh = plsc.VectorSubcoreMesh(core_axis_name="c", subcore_axis_name="s")

    @functools.partial(
        pl.kernel,
        out_type=jax.ShapeDtypeStruct((NWORKERS, NSEG), jnp.float32),
        mesh=mesh,
        scratch_types=[
            pltpu.VMEM((NSEG,), jnp.float32),
            pltpu.VMEM((CHUNK,), jnp.int32),
            pltpu.VMEM((CHUNK,), jnp.float32),
        ],
        compiler_params=pltpu.CompilerParams(needs_layout_passes=False),
    )
    def sc_kernel(cols_hbm, vals_hbm, part_hbm, acc_v, idx_v, val_v):
        cid = lax.axis_index("c")
        sid = lax.axis_index("s")
        wid = sid * 2 + cid
        base = wid * per_tile

        zeros = jnp.zeros((LANES,), jnp.float32)

        def zero_body(i, _):
            acc_v[pl.ds(i * LANES, LANES)] = zeros
            return ()

        lax.fori_loop(0, NSEG // LANES, zero_body, ())

        def chunk_body(g, _):
            off = base + g * CHUNK
            pltpu.sync_copy(cols_hbm.at[pl.ds(off, CHUNK)], idx_v)
            pltpu.sync_copy(vals_hbm.at[pl.ds(off, CHUNK)], val_v)

            def inner(i, _):
                idx = idx_v[pl.ds(i * LANES, LANES)]
                v = val_v[pl.ds(i * LANES, LANES)]
                plsc.addupdate_scatter(acc_v, [idx], v)
                return ()

            lax.fori_loop(0, CHUNK // LANES, inner, ())
            return ()

        lax.fori_loop(0, nchunk, chunk_body, ())

        pltpu.sync_copy(acc_v, part_hbm.at[wid])

    return sc_kernel(cols, vals)


def _tc_reduce(partials):
    def body(p_ref, o_ref):
        o_ref[...] = jnp.sum(p_ref[...], axis=0, keepdims=True)

    return pl.pallas_call(
        body,
        out_shape=jax.ShapeDtypeStruct((1, NSEG), jnp.float32),
    )(partials)


def kernel(indices, values):
    cols = indices[1].astype(jnp.int32)
    vals = values.astype(jnp.float32)
    nnz = vals.shape[0]

    per_tile = -(-nnz // NWORKERS)
    per_tile = -(-per_tile // CHUNK) * CHUNK
    nchunk = per_tile // CHUNK
    total = per_tile * NWORKERS

    pad = total - nnz
    cols = jnp.pad(cols, (0, pad))
    vals = jnp.pad(vals, (0, pad))

    partials = _sc_partial_sums(cols, vals, per_tile, nchunk)
    return _tc_reduce(partials).reshape(NSEG)


# speedup vs baseline: 80.8669x; 1.2658x over previous
import functools

import jax
import jax.numpy as jnp
from jax import lax
from jax.experimental import pallas as pl
from jax.experimental.pallas import tpu as pltpu
from jax.experimental.pallas import tpu_sc as plsc

NSEG = 16384
NWORKERS = 32
CHUNK = 1024
LANES = 16
NBUF = 2


def _sc_partial_sums(cols, vals, per_tile, nchunk):
    mesh = plsc.VectorSubcoreMesh(core_axis_name="c", subcore_axis_name="s")

    @functools.partial(
        pl.kernel,
        out_type=jax.ShapeDtypeStruct((NWORKERS, NSEG), jnp.float32),
        mesh=mesh,
        scratch_types=[
            pltpu.VMEM((NSEG,), jnp.float32),
            pltpu.VMEM((NBUF, CHUNK), jnp.int32),
            pltpu.VMEM((NBUF, CHUNK), jnp.float32),
            pltpu.SemaphoreType.DMA((NBUF,)),
        ],
        compiler_params=pltpu.CompilerParams(needs_layout_passes=False),
    )
    def sc_kernel(cols_hbm, vals_hbm, part_hbm, acc_v, idx_v, val_v, sem):
        cid = lax.axis_index("c")
        sid = lax.axis_index("s")
        wid = sid * 2 + cid
        base = wid * per_tile

        zeros = jnp.zeros((LANES,), jnp.float32)

        def zero_body(i, _):
            for u in range(16):
                acc_v[pl.ds(i * (16 * LANES) + u * LANES, LANES)] = zeros
            return ()

        lax.fori_loop(0, NSEG // (16 * LANES), zero_body, ())

        def start(c, b):
            off = base + c * CHUNK
            pltpu.async_copy(cols_hbm.at[pl.ds(off, CHUNK)], idx_v.at[b],
                             sem.at[b])
            pltpu.async_copy(vals_hbm.at[pl.ds(off, CHUNK)], val_v.at[b],
                             sem.at[b])

        def wait(b):
            pltpu.make_async_copy(cols_hbm.at[pl.ds(0, CHUNK)], idx_v.at[b],
                                  sem.at[b]).wait()
            pltpu.make_async_copy(vals_hbm.at[pl.ds(0, CHUNK)], val_v.at[b],
                                  sem.at[b]).wait()

        def process(b):
            for u in range(CHUNK // LANES):
                idx = idx_v[b, pl.ds(u * LANES, LANES)]
                v = val_v[b, pl.ds(u * LANES, LANES)]
                plsc.addupdate_scatter(acc_v, [idx], v)

        start(0, 0)

        def pair_body(g, _):
            for b in range(NBUF):
                c = g * NBUF + b

                @pl.when(c + 1 < nchunk)
                def _():
                    start(c + 1, (b + 1) % NBUF)

                wait(b)
                process(b)
            return ()

        lax.fori_loop(0, nchunk // NBUF, pair_body, ())

        pltpu.sync_copy(acc_v, part_hbm.at[wid])

    return sc_kernel(cols, vals)


def _tc_reduce(partials):
    def body(p_ref, o_ref):
        o_ref[...] = jnp.sum(p_ref[...], axis=0, keepdims=True)

    return pl.pallas_call(
        body,
        out_shape=jax.ShapeDtypeStruct((1, NSEG), jnp.float32),
    )(partials)


def kernel(indices, values):
    cols = indices[1].astype(jnp.int32)
    vals = values.astype(jnp.float32)
    nnz = vals.shape[0]

    step = NBUF * CHUNK
    per_tile = -(-nnz // NWORKERS)
    per_tile = -(-per_tile // step) * step
    nchunk = per_tile // CHUNK
    total = per_tile * NWORKERS

    pad = total - nnz
    cols = jnp.pad(cols, (0, pad))
    vals = jnp.pad(vals, (0, pad))

    partials = _sc_partial_sums(cols, vals, per_tile, nchunk)
    return _tc_reduce(partials).reshape(NSEG)


# speedup vs baseline: 231.5725x; 2.8636x over previous
import functools

import jax
import jax.numpy as jnp
from jax import lax
from jax.experimental import pallas as pl
from jax.experimental.pallas import tpu as pltpu
from jax.experimental.pallas import tpu_sc as plsc

NSEG = 16384
ROWS = 128
COLS = NSEG // ROWS
NWORKERS = 32
CHUNK = 2048
LANES = 16
NBUF = 2
ROUND = NWORKERS * CHUNK


def _sc_partial_sums(indices, vals, tail_cols, tail_vals, nround):
    mesh = plsc.VectorSubcoreMesh(core_axis_name="c", subcore_axis_name="s")

    @functools.partial(
        pl.kernel,
        out_type=jax.ShapeDtypeStruct((2, ROWS, COLS), jnp.float32),
        mesh=mesh,
        scratch_types=[
            pltpu.VMEM((ROWS, COLS), jnp.float32),
            pltpu.VMEM((NBUF, CHUNK), jnp.int32),
            pltpu.VMEM((NBUF, CHUNK), jnp.float32),
            pltpu.VMEM((ROWS,), jnp.int32),
            pltpu.VMEM_SHARED((ROWS, COLS), jnp.float32),
            pltpu.SemaphoreType.DMA((NBUF,)),
        ],
        compiler_params=pltpu.CompilerParams(needs_layout_passes=False),
    )
    def sc_kernel(idx_hbm, vals_hbm, tcols_hbm, tvals_hbm, part_hbm,
                  acc_v, idx_v, val_v, ridx_v, acc_s, sem):
        cid = lax.axis_index("c")
        sid = lax.axis_index("s")
        wid = sid * 2 + cid

        zeros = jnp.zeros((LANES,), jnp.float32)

        def zero_body(r, _):
            for u in range(COLS // LANES):
                acc_v[r, pl.ds(u * LANES, LANES)] = zeros
            return ()

        lax.fori_loop(0, ROWS, zero_body, ())

        @pl.when(sid == 0)
        def _():
            pltpu.sync_copy(acc_v, acc_s)

        def ridx_body(i, _):
            ridx_v[pl.ds(i * LANES, LANES)] = (
                lax.iota(jnp.int32, LANES) + i * LANES)
            return ()

        lax.fori_loop(0, ROWS // LANES, ridx_body, ())

        def start(c, b):
            off = c * NWORKERS * CHUNK + wid * CHUNK
            pltpu.async_copy(idx_hbm.at[1, pl.ds(off, CHUNK)], idx_v.at[b],
                             sem.at[b])
            pltpu.async_copy(vals_hbm.at[pl.ds(off, CHUNK)], val_v.at[b],
                             sem.at[b])

        def start_tail(b):
            off = wid * CHUNK
            pltpu.async_copy(tcols_hbm.at[pl.ds(off, CHUNK)], idx_v.at[b],
                             sem.at[b])
            pltpu.async_copy(tvals_hbm.at[pl.ds(off, CHUNK)], val_v.at[b],
                             sem.at[b])

        def wait(b):
            pltpu.make_async_copy(vals_hbm.at[pl.ds(0, CHUNK)], idx_v.at[b],
                                  sem.at[b]).wait()
            pltpu.make_async_copy(vals_hbm.at[pl.ds(0, CHUNK)], val_v.at[b],
                                  sem.at[b]).wait()

        def process(b):
            for u in range(CHUNK // LANES):
                idx = idx_v[b, pl.ds(u * LANES, LANES)]
                v = val_v[b, pl.ds(u * LANES, LANES)]
                hi = lax.shift_right_logical(idx, 7)
                lo = lax.bitwise_and(idx, jnp.int32(COLS - 1))
                plsc.addupdate_scatter(acc_v, [hi, lo], v)

        start(0, 0)

        def pair_body(g, _):
            for b in range(NBUF):
                c = g * NBUF + b

                @pl.when(c + 1 < nround)
                def _():
                    start(c + 1, (b + 1) % NBUF)

                @pl.when(c + 1 == nround)
                def _():
                    start_tail((b + 1) % NBUF)

                wait(b)
                process(b)
            return ()

        lax.fori_loop(0, nround // NBUF, pair_body, ())

        wait(0)
        process(0)

        plsc.subcore_barrier()
        pltpu.sync_copy(acc_v, acc_s.at[ridx_v], add=True)
        plsc.subcore_barrier()

        @pl.when(sid == 0)
        def _():
            pltpu.sync_copy(acc_s, part_hbm.at[cid])

    return sc_kernel(indices, vals, tail_cols, tail_vals)


def _tc_combine(partials):
    def body(p_ref, o_ref):
        o_ref[...] = p_ref[0] + p_ref[1]

    return pl.pallas_call(
        body,
        out_shape=jax.ShapeDtypeStruct((ROWS, COLS), jnp.float32),
    )(partials)


def kernel(indices, values):
    indices = indices.astype(jnp.int32)
    vals = values.astype(jnp.float32)
    nnz = vals.shape[0]

    nround = nnz // ROUND
    bulk = nround * ROUND
    tail = nnz - bulk

    pad = ROUND - tail
    tail_cols = jnp.pad(indices[1, bulk:], (0, pad))
    tail_vals = jnp.pad(vals[bulk:], (0, pad))

    partials = _sc_partial_sums(indices, vals, tail_cols, tail_vals, nround)
    return _tc_combine(partials).reshape(NSEG)


# speedup vs baseline: 232.9556x; 1.0060x over previous
import functools

import jax
import jax.numpy as jnp
from jax import lax
from jax.experimental import pallas as pl
from jax.experimental.pallas import tpu as pltpu
from jax.experimental.pallas import tpu_sc as plsc

NSEG = 16384
ROWS = 128
COLS = NSEG // ROWS
NWORKERS = 32
CHUNK = 2048
LANES = 16
NBUF = 2
ROUND = NWORKERS * CHUNK


def _sc_partial_sums(indices, vals, tail_cols, tail_vals, nround):
    mesh = plsc.VectorSubcoreMesh(core_axis_name="c", subcore_axis_name="s")

    @functools.partial(
        pl.kernel,
        out_type=jax.ShapeDtypeStruct((2, ROWS, COLS), jnp.float32),
        mesh=mesh,
        scratch_types=[
            pltpu.VMEM((ROWS, COLS), jnp.float32),
            pltpu.VMEM((NBUF, CHUNK), jnp.int32),
            pltpu.VMEM((NBUF, CHUNK), jnp.float32),
            pltpu.VMEM((ROWS,), jnp.int32),
            pltpu.VMEM_SHARED((ROWS, COLS), jnp.float32),
            pltpu.SemaphoreType.DMA((NBUF,)),
        ],
        compiler_params=pltpu.CompilerParams(
            needs_layout_passes=False,
            disable_bounds_checks=True,
        ),
    )
    def sc_kernel(idx_hbm, vals_hbm, tcols_hbm, tvals_hbm, part_hbm,
                  acc_v, idx_v, val_v, ridx_v, acc_s, sem):
        cid = lax.axis_index("c")
        sid = lax.axis_index("s")
        wid = sid * 2 + cid

        def start(c, b):
            off = c * NWORKERS * CHUNK + wid * CHUNK
            pltpu.async_copy(idx_hbm.at[1, pl.ds(off, CHUNK)], idx_v.at[b],
                             sem.at[b])
            pltpu.async_copy(vals_hbm.at[pl.ds(off, CHUNK)], val_v.at[b],
                             sem.at[b])

        def start_tail(b):
            off = wid * CHUNK
            pltpu.async_copy(tcols_hbm.at[pl.ds(off, CHUNK)], idx_v.at[b],
                             sem.at[b])
            pltpu.async_copy(tvals_hbm.at[pl.ds(off, CHUNK)], val_v.at[b],
                             sem.at[b])

        def wait(b):
            pltpu.make_async_copy(vals_hbm.at[pl.ds(0, CHUNK)], idx_v.at[b],
                                  sem.at[b]).wait()
            pltpu.make_async_copy(vals_hbm.at[pl.ds(0, CHUNK)], val_v.at[b],
                                  sem.at[b]).wait()

        start(0, 0)
        start(1, 1)

        zeros = jnp.zeros((LANES,), jnp.float32)

        def zero_body(r, _):
            for u in range(COLS // LANES):
                acc_v[r, pl.ds(u * LANES, LANES)] = zeros
            return ()

        lax.fori_loop(0, ROWS, zero_body, ())

        @pl.when(sid == 0)
        def _():
            pltpu.sync_copy(acc_v, acc_s)

        def ridx_body(i, _):
            ridx_v[pl.ds(i * LANES, LANES)] = (
                lax.iota(jnp.int32, LANES) + i * LANES)
            return ()

        lax.fori_loop(0, ROWS // LANES, ridx_body, ())

        def process(b):
            for u in range(CHUNK // LANES):
                idx = idx_v[b, pl.ds(u * LANES, LANES)]
                v = val_v[b, pl.ds(u * LANES, LANES)]
                hi = lax.shift_right_logical(idx, 7)
                lo = lax.bitwise_and(idx, jnp.int32(COLS - 1))
                plsc.addupdate_scatter(acc_v, [hi, lo], v)

        def pair_body2(g, _):
            for b in range(NBUF):
                c = g * NBUF + b
                wait(b)
                process(b)

                @pl.when(c + 2 < nround)
                def _():
                    start(c + 2, b)

                @pl.when(c + 2 == nround)
                def _():
                    start_tail(b)
            return ()

        lax.fori_loop(0, nround // NBUF, pair_body2, ())

        wait(0)
        process(0)

        plsc.subcore_barrier()
        pltpu.sync_copy(acc_v, acc_s.at[ridx_v], add=True)
        plsc.subcore_barrier()

        @pl.when(sid == 0)
        def _():
            pltpu.sync_copy(acc_s, part_hbm.at[cid])

    return sc_kernel(indices, vals, tail_cols, tail_vals)


def _tc_combine(partials):
    def body(p_ref, o_ref):
        o_ref[...] = p_ref[0] + p_ref[1]

    return pl.pallas_call(
        body,
        out_shape=jax.ShapeDtypeStruct((ROWS, COLS), jnp.float32),
    )(partials)


def kernel(indices, values):
    indices = indices.astype(jnp.int32)
    vals = values.astype(jnp.float32)
    nnz = vals.shape[0]

    nround = nnz // ROUND
    bulk = nround * ROUND
    tail = nnz - bulk

    pad = ROUND - tail
    tail_cols = jnp.pad(indices[1, bulk:], (0, pad))
    tail_vals = jnp.pad(vals[bulk:], (0, pad))

    partials = _sc_partial_sums(indices, vals, tail_cols, tail_vals, nround)
    return _tc_combine(partials).reshape(NSEG)


# speedup vs baseline: 291.5290x; 1.2514x over previous
import functools

import jax
import jax.numpy as jnp
from jax import lax
from jax.experimental import pallas as pl
from jax.experimental.pallas import tpu as pltpu
from jax.experimental.pallas import tpu_sc as plsc

NSEG = 16384
ROWS = 128
COLS = NSEG // ROWS
NWORKERS = 32
CHUNK = 2048
LANES = 16
NBUF = 2
ROUND = NWORKERS * CHUNK


def _sc_partial_sums(indices, vals, tail_cols, tail_vals, nround):
    mesh = plsc.VectorSubcoreMesh(core_axis_name="c", subcore_axis_name="s")

    @functools.partial(
        pl.kernel,
        out_type=jax.ShapeDtypeStruct((2, ROWS, COLS), jnp.float32),
        mesh=mesh,
        scratch_types=[
            pltpu.VMEM((ROWS, COLS), jnp.float32),
            pltpu.VMEM((NBUF, CHUNK), jnp.int32),
            pltpu.VMEM((NBUF, CHUNK), jnp.float32),
            pltpu.VMEM((ROWS,), jnp.int32),
            pltpu.VMEM_SHARED((ROWS, COLS), jnp.float32),
            pltpu.SemaphoreType.DMA((NBUF,)),
        ],
        compiler_params=pltpu.CompilerParams(
            needs_layout_passes=False,
            disable_bounds_checks=True,
        ),
    )
    def sc_kernel(idx_hbm, vals_hbm, tcols_hbm, tvals_hbm, part_hbm,
                  acc_v, idx_v, val_v, ridx_v, acc_s, sem):
        cid = lax.axis_index("c")
        sid = lax.axis_index("s")
        wid = sid * 2 + cid

        def start(c, b):
            off = c * NWORKERS * CHUNK + wid * CHUNK
            pltpu.async_copy(idx_hbm.at[1, pl.ds(off, CHUNK)], idx_v.at[b],
                             sem.at[b])
            pltpu.async_copy(vals_hbm.at[pl.ds(off, CHUNK)], val_v.at[b],
                             sem.at[b])

        def start_tail(b):
            off = wid * CHUNK
            pltpu.async_copy(tcols_hbm.at[pl.ds(off, CHUNK)], idx_v.at[b],
                             sem.at[b])
            pltpu.async_copy(tvals_hbm.at[pl.ds(off, CHUNK)], val_v.at[b],
                             sem.at[b])

        def wait(b):
            pltpu.make_async_copy(vals_hbm.at[pl.ds(0, CHUNK)], idx_v.at[b],
                                  sem.at[b]).wait()
            pltpu.make_async_copy(vals_hbm.at[pl.ds(0, CHUNK)], val_v.at[b],
                                  sem.at[b]).wait()

        start(0, 0)
        start(1, 1)

        zeros = jnp.zeros((LANES,), jnp.float32)

        def zero_body(r, _):
            for u in range(COLS // LANES):
                acc_v[r, pl.ds(u * LANES, LANES)] = zeros
            return ()

        lax.fori_loop(0, ROWS, zero_body, ())

        @pl.when(sid == 0)
        def _():
            pltpu.sync_copy(acc_v, acc_s)

        def ridx_body(i, _):
            ridx_v[pl.ds(i * LANES, LANES)] = (
                lax.iota(jnp.int32, LANES) + i * LANES)
            return ()

        lax.fori_loop(0, ROWS // LANES, ridx_body, ())

        def process(b):
            @plsc.parallel_loop(0, CHUNK, LANES, unroll=16)
            def _(o):
                idx = idx_v[b, pl.ds(o, LANES)]
                v = val_v[b, pl.ds(o, LANES)]
                hi = lax.shift_right_logical(idx, 7)
                lo = lax.bitwise_and(idx, jnp.int32(COLS - 1))
                plsc.addupdate_scatter(acc_v, [hi, lo], v)

        def pair_body2(g, _):
            for b in range(NBUF):
                c = g * NBUF + b
                wait(b)
                process(b)

                @pl.when(c + 2 < nround)
                def _():
                    start(c + 2, b)

                @pl.when(c + 2 == nround)
                def _():
                    start_tail(b)
            return ()

        lax.fori_loop(0, nround // NBUF, pair_body2, ())

        wait(0)
        process(0)

        plsc.subcore_barrier()
        pltpu.sync_copy(acc_v, acc_s.at[ridx_v], add=True)
        plsc.subcore_barrier()

        @pl.when(sid == 0)
        def _():
            pltpu.sync_copy(acc_s, part_hbm.at[cid])

    return sc_kernel(indices, vals, tail_cols, tail_vals)


def _tc_combine(partials):
    def body(p_ref, o_ref):
        o_ref[...] = p_ref[0] + p_ref[1]

    return pl.pallas_call(
        body,
        out_shape=jax.ShapeDtypeStruct((ROWS, COLS), jnp.float32),
    )(partials)


def kernel(indices, values):
    indices = indices.astype(jnp.int32)
    vals = values.astype(jnp.float32)
    nnz = vals.shape[0]

    nround = nnz // ROUND
    bulk = nround * ROUND
    tail = nnz - bulk

    pad = ROUND - tail
    tail_cols = jnp.pad(indices[1, bulk:], (0, pad))
    tail_vals = jnp.pad(vals[bulk:], (0, pad))

    partials = _sc_partial_sums(indices, vals, tail_cols, tail_vals, nround)
    return _tc_combine(partials).reshape(NSEG)


# speedup vs baseline: 292.6181x; 1.0037x over previous
import functools

import jax
import jax.numpy as jnp
from jax import lax
from jax.experimental import pallas as pl
from jax.experimental.pallas import tpu as pltpu
from jax.experimental.pallas import tpu_sc as plsc

NSEG = 16384
ROWS = 128
COLS = NSEG // ROWS
NWORKERS = 32
CHUNK = 2048
LANES = 16
NBUF = 2
ROUND = NWORKERS * CHUNK


def _sc_partial_sums(indices, vals, tail_cols, tail_vals, nround):
    mesh = plsc.VectorSubcoreMesh(core_axis_name="c", subcore_axis_name="s")

    @functools.partial(
        pl.kernel,
        out_type=jax.ShapeDtypeStruct((2, ROWS, COLS), jnp.float32),
        mesh=mesh,
        scratch_types=[
            pltpu.VMEM((ROWS, COLS), jnp.float32),
            pltpu.VMEM((NBUF, CHUNK), jnp.int32),
            pltpu.VMEM((NBUF, CHUNK), jnp.float32),
            pltpu.VMEM((ROWS,), jnp.int32),
            pltpu.VMEM_SHARED((ROWS, COLS), jnp.float32),
            pltpu.SemaphoreType.DMA((NBUF,)),
        ],
        compiler_params=pltpu.CompilerParams(
            needs_layout_passes=False,
            disable_bounds_checks=True,
        ),
    )
    def sc_kernel(idx_hbm, vals_hbm, tcols_hbm, tvals_hbm, part_hbm,
                  acc_v, idx_v, val_v, ridx_v, acc_s, sem):
        cid = lax.axis_index("c")
        sid = lax.axis_index("s")
        wid = sid * 2 + cid

        def start(c, b):
            off = c * NWORKERS * CHUNK + wid * CHUNK
            pltpu.async_copy(idx_hbm.at[1, pl.ds(off, CHUNK)], idx_v.at[b],
                             sem.at[b])
            pltpu.async_copy(vals_hbm.at[pl.ds(off, CHUNK)], val_v.at[b],
                             sem.at[b])

        def start_tail(b):
            off = wid * CHUNK
            pltpu.async_copy(tcols_hbm.at[pl.ds(off, CHUNK)], idx_v.at[b],
                             sem.at[b])
            pltpu.async_copy(tvals_hbm.at[pl.ds(off, CHUNK)], val_v.at[b],
                             sem.at[b])

        def wait(b):
            pltpu.make_async_copy(vals_hbm.at[pl.ds(0, CHUNK)], idx_v.at[b],
                                  sem.at[b]).wait()
            pltpu.make_async_copy(vals_hbm.at[pl.ds(0, CHUNK)], val_v.at[b],
                                  sem.at[b]).wait()

        start(0, 0)
        start(1, 1)

        zeros = jnp.zeros((LANES,), jnp.float32)

        def zero_body(r, _):
            for u in range(COLS // LANES):
                acc_v[r, pl.ds(u * LANES, LANES)] = zeros
            return ()

        lax.fori_loop(0, ROWS, zero_body, ())

        @pl.when(sid == 0)
        def _():
            pltpu.sync_copy(acc_v, acc_s)

        def ridx_body(i, _):
            ridx_v[pl.ds(i * LANES, LANES)] = (
                lax.iota(jnp.int32, LANES) + i * LANES)
            return ()

        lax.fori_loop(0, ROWS // LANES, ridx_body, ())

        def process(b):
            @plsc.parallel_loop(0, CHUNK, LANES, unroll=8)
            def _(o):
                idx = idx_v[b, pl.ds(o, LANES)]
                v = val_v[b, pl.ds(o, LANES)]
                hi = lax.shift_right_logical(idx, 7)
                lo = lax.bitwise_and(idx, jnp.int32(COLS - 1))
                plsc.addupdate_scatter(acc_v, [hi, lo], v)

        def pair_body2(g, _):
            for b in range(NBUF):
                c = g * NBUF + b
                wait(b)
                process(b)

                @pl.when(c + 2 < nround)
                def _():
                    start(c + 2, b)

                @pl.when(c + 2 == nround)
                def _():
                    start_tail(b)
            return ()

        lax.fori_loop(0, nround // NBUF, pair_body2, ())

        wait(0)
        process(0)

        plsc.subcore_barrier()
        pltpu.sync_copy(acc_v, acc_s.at[ridx_v], add=True)
        plsc.subcore_barrier()

        @pl.when(sid == 0)
        def _():
            pltpu.sync_copy(acc_s, part_hbm.at[cid])

    return sc_kernel(indices, vals, tail_cols, tail_vals)


def _tc_combine(partials):
    def body(p_ref, o_ref):
        o_ref[...] = p_ref[0] + p_ref[1]

    return pl.pallas_call(
        body,
        out_shape=jax.ShapeDtypeStruct((ROWS, COLS), jnp.float32),
    )(partials)


def kernel(indices, values):
    indices = indices.astype(jnp.int32)
    vals = values.astype(jnp.float32)
    nnz = vals.shape[0]

    nround = nnz // ROUND
    bulk = nround * ROUND
    tail = nnz - bulk

    pad = ROUND - tail
    tail_cols = jnp.pad(indices[1, bulk:], (0, pad))
    tail_vals = jnp.pad(vals[bulk:], (0, pad))

    partials = _sc_partial_sums(indices, vals, tail_cols, tail_vals, nround)
    return _tc_combine(partials).reshape(NSEG)
